# baseline (speedup 1.0000x reference)
import functools

import jax
import jax.numpy as jnp
from jax import lax
from jax.experimental import pallas as pl
from jax.experimental.pallas import tpu as pltpu
from jax.experimental.pallas import tpu_sc as plsc

_NC = 2
_NS = 16
_NW = _NC * _NS
_NBUF = 4


@functools.lru_cache(maxsize=None)
def _build(batch, hist, vocab, d):
    rows_per_tile = batch // _NW
    ch0 = (hist // 2) // 8 * 8
    ch1 = hist - ch0
    n_ch = rows_per_tile * 2
    mesh = plsc.VectorSubcoreMesh(core_axis_name="c", subcore_axis_name="s")

    def chunk(i, par):
        return i // 2, par * ch0, ch1 if par else ch0

    @functools.partial(
        pl.kernel,
        mesh=mesh,
        out_type=jax.ShapeDtypeStruct((batch, hist, d), jnp.float32),
        scratch_types=[
            pltpu.VMEM((rows_per_tile, hist), jnp.int32),
            pltpu.VMEM((_NBUF, ch1, d), jnp.float32),
            pltpu.SemaphoreType.DMA((_NBUF,)),
            pltpu.SemaphoreType.DMA((_NBUF,)),
        ],
        compiler_params=pltpu.CompilerParams(use_tc_tiling_on_sc=False),
    )
    def gather_kernel(x_hbm, table_hbm, out_hbm, xv, rows_v, gsem, osem):
        wid = lax.axis_index("s") * _NC + lax.axis_index("c")
        b0 = wid * rows_per_tile
        pltpu.sync_copy(x_hbm.at[pl.ds(b0, rows_per_tile)], xv)

        def gather_copy(i, b, par):
            r, h0, ln = chunk(i, par)
            return pltpu.make_async_copy(
                table_hbm.at[xv.at[r, pl.ds(h0, ln)]],
                rows_v.at[b, pl.ds(0, ln)], gsem.at[b])

        def out_copy(i, b, par):
            r, h0, ln = chunk(i, par)
            return pltpu.make_async_copy(
                rows_v.at[b, pl.ds(0, ln)],
                out_hbm.at[b0 + r, pl.ds(h0, ln), :], osem.at[b])

        gather_copy(0, 0, 0).start()
        gather_copy(1, 1, 1).start()
        for i in range(2):
            gather_copy(i, i, i % 2).wait()
            out_copy(i, i, i % 2).start()
            gather_copy(i + 2, i + 2, i % 2).start()

        def block(p, carry):
            i0 = 2 + p * _NBUF
            for dlt in range(_NBUF):
                i = i0 + dlt
                b = (2 + dlt) % _NBUF
                b2 = dlt % _NBUF
                par = dlt % 2
                gather_copy(i, b, par).wait()
                out_copy(i, b, par).start()
                out_copy(i - 2, b2, par).wait()
                gather_copy(i + 2, b2, par).start()
            return carry

        lax.fori_loop(0, (n_ch - 4) // _NBUF, block, 0)

        for i in range(n_ch - 2, n_ch):
            gather_copy(i, i % _NBUF, i % 2).wait()
            out_copy(i, i % _NBUF, i % 2).start()
        for i in range(n_ch - 4, n_ch):
            out_copy(i, i % _NBUF, i % 2).wait()

    return gather_kernel


def kernel(x, embedding):
    batch, hist = x.shape
    vocab, d = embedding.shape
    return _build(batch, hist, vocab, d)(x.astype(jnp.int32), embedding)
